# baseline (speedup 1.0000x reference)
import jax
import jax.numpy as jnp
from jax import lax
from jax.experimental import pallas as pl
from jax.experimental.pallas import tpu as pltpu
from jax.experimental.pallas import tpu_sc as plsc

B = 16384
D = 128
K = 16
NUM_REL = 64

NC = 2
NS = 16
NW = NC * NS
PER_W = B // NW
C = 16
N_CHUNKS = PER_W // C
JC = C * K // 128

_f32 = jnp.float32
_i32 = jnp.int32


def _sc1_body(u_hbm, adj_idx_hbm, uandi_hbm, usr_hbm, ent_hbm, adj_ent_hbm,
              adj_rel_hbm, wb_hbm, t_out, ne_out, nr_out,
              u_all, uandi_all, adjidx_all, ne_all, nr_all, ustage, rows_v,
              user_v, wb_v, t_v, semA, semB, semOutA, semOutB, semAdj):
    wid = lax.axis_index("s") * NC + lax.axis_index("c")
    wbase = pl.multiple_of(wid * PER_W, PER_W)
    pltpu.sync_copy(wb_hbm, wb_v)
    for j in range(PER_W // 128):
        pltpu.sync_copy(u_hbm.at[pl.ds(wbase + j * 128, 128)], u_all.at[j])
    for j in range(PER_W * K // 128):
        pltpu.sync_copy(uandi_hbm.at[pl.ds(wbase * K + j * 128, 128)],
                        uandi_all.at[j])
        pltpu.sync_copy(adj_idx_hbm.at[pl.ds(wbase * K + j * 128, 128)],
                        adjidx_all.at[j])
    adj_cps = []
    for j in range(PER_W * K // 128):
        adj_cps.append(pltpu.async_copy(adj_ent_hbm.at[adjidx_all.at[j]],
                                        ne_all.at[j], semAdj))
        adj_cps.append(pltpu.async_copy(adj_rel_hbm.at[adjidx_all.at[j]],
                                        nr_all.at[j], semAdj))

    def fire_rows(c, buf, sem):
        for j in range(JC):
            pltpu.async_copy(ent_hbm.at[uandi_all.at[c * JC + j]],
                             rows_v.at[buf, pl.ds(j * 128, 128)], sem)
        ustage[buf] = u_all[c // 8, pl.ds((c % 8) * 16, 16)]
        pltpu.async_copy(usr_hbm.at[ustage.at[buf]], user_v.at[buf], sem)

    def wait_rows(c, buf, sem):
        for j in range(JC):
            pltpu.make_async_copy(ent_hbm.at[uandi_all.at[c * JC + j]],
                                  rows_v.at[buf, pl.ds(j * 128, 128)],
                                  sem).wait()
        pltpu.make_async_copy(usr_hbm.at[ustage.at[buf]],
                              user_v.at[buf], sem).wait()

    def compute(buf):
        ws = [[wb_v[(3 * c + k) // 8, pl.ds(((3 * c + k) % 8) * 16, 16)]
               for k in range(3)] for c in range(K)]

        def belem(b, carry2):
            accs = [[jnp.zeros((16,), _f32) for _ in range(8)]
                    for _ in range(3)]
            for c in range(K):
                w0, w1, w2 = ws[c]
                for dc in range(8):
                    r = rows_v[buf, b * K + c, pl.ds(dc * 16, 16)]
                    accs[0][dc] += r * w0
                    accs[1][dc] += r * w1
                    accs[2][dc] += r * w2
            for dc in range(8):
                ur = user_v[buf, b, pl.ds(dc * 16, 16)]
                for k in range(3):
                    t_v[buf, k, b, pl.ds(dc * 16, 16)] = accs[k][dc] * ur
            return carry2

        lax.fori_loop(0, C, belem, 0)

    def fire_out(c, buf, sem):
        base = pl.multiple_of(wbase + c * C, C)
        for k in range(3):
            pltpu.async_copy(t_v.at[buf, k], t_out.at[k, pl.ds(base, C)], sem)

    def wait_out(c, buf, sem):
        base = pl.multiple_of(wbase + c * C, C)
        for k in range(3):
            pltpu.make_async_copy(t_v.at[buf, k],
                                  t_out.at[k, pl.ds(base, C)], sem).wait()

    fire_rows(0, 0, semA)

    def pipe(i, carry):
        cA = i * 2
        cB = i * 2 + 1
        fire_rows(cB, 1, semB)

        @pl.when(i > 0)
        def _():
            wait_out(cA - 2, 0, semOutA)

        wait_rows(cA, 0, semA)
        compute(0)
        fire_out(cA, 0, semOutA)

        @pl.when(i < N_CHUNKS // 2 - 1)
        def _():
            fire_rows(cB + 1, 0, semA)

        @pl.when(i > 0)
        def _():
            wait_out(cB - 2, 1, semOutB)

        wait_rows(cB, 1, semB)
        compute(1)
        fire_out(cB, 1, semOutB)
        return carry

    lax.fori_loop(0, N_CHUNKS // 2, pipe, 0)
    wait_out(N_CHUNKS - 2, 0, semOutA)
    wait_out(N_CHUNKS - 1, 1, semOutB)
    for cp in adj_cps:
        cp.wait()
    for j in range(PER_W * K // 128):
        pltpu.sync_copy(ne_all.at[j],
                        ne_out.at[pl.ds(wbase * K + j * 128, 128)])
        pltpu.sync_copy(nr_all.at[j],
                        nr_out.at[pl.ds(wbase * K + j * 128, 128)])


def _sc_stage1(u, adj_idx, uandi_flat, usr_table, ent_table, adj_ent_flat,
               adj_rel_flat, wb):
    mesh = plsc.VectorSubcoreMesh(core_axis_name="c", subcore_axis_name="s")
    kfn = pl.kernel(
        _sc1_body,
        out_type=(
            jax.ShapeDtypeStruct((3, B, D), _f32),
            jax.ShapeDtypeStruct((B * K,), _i32),
            jax.ShapeDtypeStruct((B * K,), _i32),
        ),
        mesh=mesh,
        scratch_types=[
            pltpu.VMEM((PER_W // 128, 128), _i32),
            pltpu.VMEM((PER_W * K // 128, 128), _i32),
            pltpu.VMEM((PER_W * K // 128, 128), _i32),
            pltpu.VMEM((PER_W * K // 128, 128), _i32),
            pltpu.VMEM((PER_W * K // 128, 128), _i32),
            pltpu.VMEM((2, 16), _i32),
            pltpu.VMEM((2, C * K, D), _f32),
            pltpu.VMEM((2, C, D), _f32),
            pltpu.VMEM((6, 128), _f32),
            pltpu.VMEM((2, 3, C, D), _f32),
            pltpu.SemaphoreType.DMA,
            pltpu.SemaphoreType.DMA,
            pltpu.SemaphoreType.DMA,
            pltpu.SemaphoreType.DMA,
            pltpu.SemaphoreType.DMA,
        ],
    )
    return kfn(u, adj_idx, uandi_flat, usr_table, ent_table, adj_ent_flat,
               adj_rel_flat, wb)


def _sc3_body(v_hbm, ne_hbm, sc_hbm, ent_hbm, agg_out,
              v_all, ne_all, vstage, rows_v, self_v, sc_v, out_v,
              semA, semB, semOutA, semOutB):
    wid = lax.axis_index("s") * NC + lax.axis_index("c")
    wbase = pl.multiple_of(wid * PER_W, PER_W)
    for j in range(PER_W // 128):
        pltpu.sync_copy(v_hbm.at[pl.ds(wbase + j * 128, 128)], v_all.at[j])
    for j in range(PER_W * K // 128):
        pltpu.sync_copy(ne_hbm.at[pl.ds(wbase * K + j * 128, 128)],
                        ne_all.at[j])

    def fire_rows(c, buf, sem):
        for j in range(JC):
            pltpu.async_copy(ent_hbm.at[ne_all.at[c * JC + j]],
                             rows_v.at[buf, pl.ds(j * 128, 128)], sem)
        vstage[buf] = v_all[c // 8, pl.ds((c % 8) * 16, 16)]
        pltpu.async_copy(ent_hbm.at[vstage.at[buf]], self_v.at[buf], sem)
        base = pl.multiple_of(wbase + c * C, C)
        for h in range(2):
            pltpu.async_copy(sc_hbm.at[h, pl.ds(base, C)],
                             sc_v.at[buf, h], sem)

    def wait_rows(c, buf, sem):
        for j in range(JC):
            pltpu.make_async_copy(ent_hbm.at[ne_all.at[c * JC + j]],
                                  rows_v.at[buf, pl.ds(j * 128, 128)],
                                  sem).wait()
        pltpu.make_async_copy(ent_hbm.at[vstage.at[buf]],
                              self_v.at[buf], sem).wait()
        base = pl.multiple_of(wbase + c * C, C)
        for h in range(2):
            pltpu.make_async_copy(sc_hbm.at[h, pl.ds(base, C)],
                                  sc_v.at[buf, h], sem).wait()

    def compute(buf):
        def belem(b, carry2):
            accs = [self_v[buf, b, pl.ds(dc * 16, 16)] for dc in range(8)]
            for k in range(K):
                s = sc_v[buf, k // 8, b, pl.ds((k % 8) * 16, 16)]
                for dc in range(8):
                    accs[dc] += rows_v[buf, b * K + k, pl.ds(dc * 16, 16)] * s
            for dc in range(8):
                out_v[buf, b, pl.ds(dc * 16, 16)] = accs[dc]
            return carry2

        lax.fori_loop(0, C, belem, 0)

    def fire_out(c, buf, sem):
        base = pl.multiple_of(wbase + c * C, C)
        pltpu.async_copy(out_v.at[buf], agg_out.at[pl.ds(base, C)], sem)

    def wait_out(c, buf, sem):
        base = pl.multiple_of(wbase + c * C, C)
        pltpu.make_async_copy(out_v.at[buf],
                              agg_out.at[pl.ds(base, C)], sem).wait()

    fire_rows(0, 0, semA)

    def pipe(i, carry):
        cA = i * 2
        cB = i * 2 + 1
        fire_rows(cB, 1, semB)

        @pl.when(i > 0)
        def _():
            wait_out(cA - 2, 0, semOutA)

        wait_rows(cA, 0, semA)
        compute(0)
        fire_out(cA, 0, semOutA)

        @pl.when(i < N_CHUNKS // 2 - 1)
        def _():
            fire_rows(cB + 1, 0, semA)

        @pl.when(i > 0)
        def _():
            wait_out(cB - 2, 1, semOutB)

        wait_rows(cB, 1, semB)
        compute(1)
        fire_out(cB, 1, semOutB)
        return carry

    lax.fori_loop(0, N_CHUNKS // 2, pipe, 0)
    wait_out(N_CHUNKS - 2, 0, semOutA)
    wait_out(N_CHUNKS - 1, 1, semOutB)


def _sc_stage3(v, ne_flat, scores, ent_table):
    mesh = plsc.VectorSubcoreMesh(core_axis_name="c", subcore_axis_name="s")
    kfn = pl.kernel(
        _sc3_body,
        out_type=jax.ShapeDtypeStruct((B, D), _f32),
        mesh=mesh,
        scratch_types=[
            pltpu.VMEM((PER_W // 128, 128), _i32),
            pltpu.VMEM((PER_W * K // 128, 128), _i32),
            pltpu.VMEM((2, 16), _i32),
            pltpu.VMEM((2, C * K, D), _f32),
            pltpu.VMEM((2, C, D), _f32),
            pltpu.VMEM((2, 2, C, D), _f32),
            pltpu.VMEM((2, C, D), _f32),
            pltpu.SemaphoreType.DMA,
            pltpu.SemaphoreType.DMA,
            pltpu.SemaphoreType.DMA,
            pltpu.SemaphoreType.DMA,
        ],
    )
    return kfn(v, ne_flat, scores, ent_table)


R2 = 512


def _tc2_body(t_ref, nbr_ref, rel_ref, cb_ref, ue_ref, sc_ref):
    t0 = t_ref[0]
    t1 = t_ref[1]
    t2 = t_ref[2]
    z = jnp.zeros((R2, 1), _f32)
    ue = (jnp.concatenate([z, t0[:, :-1]], axis=1) + t1
          + jnp.concatenate([t2[:, 1:], z], axis=1) + cb_ref[0, 0])
    ue_ref[...] = ue
    p = lax.dot_general(ue, rel_ref[...], (((1,), (1,)), ((), ())),
                        preferred_element_type=_f32)
    nbr = nbr_ref[...]
    iota = lax.broadcasted_iota(_i32, (R2, NUM_REL), 1)
    cols = []
    for k in range(K):
        masked = jnp.where(nbr[:, k:k + 1] == iota, p, 0.0)
        cols.append(jnp.sum(masked, axis=1, keepdims=True))
    scores = jnp.concatenate(cols, axis=1)
    m = jnp.max(scores, axis=1, keepdims=True)
    e = jnp.exp(scores - m)
    sm = e / jnp.sum(e, axis=1, keepdims=True)
    lane_k = lax.broadcasted_iota(_i32, (K, D), 1) // 16
    row_k = lax.broadcasted_iota(_i32, (K, D), 0)
    e0 = (row_k == lane_k).astype(_f32)
    e1 = (row_k == lane_k + 8).astype(_f32)
    sc_ref[0] = jnp.dot(sm, e0, preferred_element_type=_f32)
    sc_ref[1] = jnp.dot(sm, e1, preferred_element_type=_f32)


def _tc_stage2(t, nbr_rel, rel_table, conv_b):
    grid = (B // R2,)
    return pl.pallas_call(
        _tc2_body,
        grid=grid,
        in_specs=[
            pl.BlockSpec((3, R2, D), lambda i: (0, i, 0)),
            pl.BlockSpec((R2, K), lambda i: (i, 0)),
            pl.BlockSpec((NUM_REL, D), lambda i: (0, 0)),
            pl.BlockSpec((1, 1), lambda i: (0, 0)),
        ],
        out_specs=[
            pl.BlockSpec((R2, D), lambda i: (i, 0)),
            pl.BlockSpec((2, R2, D), lambda i: (0, i, 0)),
        ],
        out_shape=[
            jax.ShapeDtypeStruct((B, D), _f32),
            jax.ShapeDtypeStruct((2, B, D), _f32),
        ],
    )(t, nbr_rel, rel_table, conv_b)


R4 = 512


def _tc4_body(agg_ref, ue_ref, w_ref, b_ref, out_ref):
    item = jnp.tanh(
        jnp.dot(agg_ref[...], w_ref[...], preferred_element_type=_f32)
        + b_ref[...])
    s = jnp.sum(ue_ref[...] * item, axis=1)
    out_ref[...] = 1.0 / (1.0 + jnp.exp(-s))


def _tc_stage4(agg, ue, agg_w, agg_b):
    grid = (B // R4,)
    return pl.pallas_call(
        _tc4_body,
        grid=grid,
        in_specs=[
            pl.BlockSpec((R4, D), lambda i: (i, 0)),
            pl.BlockSpec((R4, D), lambda i: (i, 0)),
            pl.BlockSpec((D, D), lambda i: (0, 0)),
            pl.BlockSpec((1, D), lambda i: (0, 0)),
        ],
        out_specs=pl.BlockSpec((R4,), lambda i: (i,)),
        out_shape=jax.ShapeDtypeStruct((B,), _f32),
    )(agg, ue, agg_w, agg_b)


@jax.jit
def kernel(u, v, uandi_adj, usr_table, ent_table, rel_table, adj_ent,
           adj_rel, conv_w, conv_b, agg_w, agg_b):
    u = u.astype(_i32)
    v = v.astype(_i32)
    uandi_flat = uandi_adj.astype(_i32).reshape(-1)
    adj_ent = adj_ent.astype(_i32)
    adj_rel = adj_rel.astype(_i32)

    w_eff = conv_w[0, :, :, 1]
    wb = jnp.broadcast_to(w_eff.reshape(6, 8, 1),
                          (6, 8, 16)).reshape(6, 128)

    adj_idx = (v[:, None] * K + jnp.arange(K, dtype=_i32)).reshape(-1)

    t, ne, nr = _sc_stage1(u, adj_idx, uandi_flat, usr_table, ent_table,
                           adj_ent.reshape(-1), adj_rel.reshape(-1), wb)
    ue, scores = _tc_stage2(t, nr.reshape(B, K), rel_table,
                            conv_b.reshape(1, 1))
    agg = _sc_stage3(v, ne, scores, ent_table)
    return _tc_stage4(agg, ue, agg_w, agg_b.reshape(1, D))


# speedup vs baseline: 1.0090x; 1.0090x over previous
import jax
import jax.numpy as jnp
from jax import lax
from jax.experimental import pallas as pl
from jax.experimental.pallas import tpu as pltpu
from jax.experimental.pallas import tpu_sc as plsc

B = 16384
D = 128
K = 16
NUM_REL = 64

NC = 2
NS = 16
NW = NC * NS
PER_W = B // NW
C = 16
N_CHUNKS = PER_W // C
JC = C * K // 128

_f32 = jnp.float32
_i32 = jnp.int32


def _sc1_body(u_hbm, adj_idx_hbm, uandi_hbm, usr_hbm, ent_hbm, adj_ent_hbm,
              adj_rel_hbm, wb_hbm, t_out, ne_out, nr_out,
              u_all, uandi_all, adjidx_all, ne_all, nr_all, ustage, rows_v,
              user_v, wb_v, t_v, semA, semB, semOutA, semOutB, semAdj):
    wid = lax.axis_index("s") * NC + lax.axis_index("c")
    wbase = pl.multiple_of(wid * PER_W, PER_W)
    pltpu.sync_copy(wb_hbm, wb_v)
    for j in range(PER_W // 128):
        pltpu.sync_copy(u_hbm.at[pl.ds(wbase + j * 128, 128)], u_all.at[j])
    for j in range(PER_W * K // 128):
        pltpu.sync_copy(uandi_hbm.at[pl.ds(wbase * K + j * 128, 128)],
                        uandi_all.at[j])
        pltpu.sync_copy(adj_idx_hbm.at[pl.ds(wbase * K + j * 128, 128)],
                        adjidx_all.at[j])
    adj_cps = []
    for j in range(PER_W * K // 128):
        adj_cps.append(pltpu.async_copy(adj_ent_hbm.at[adjidx_all.at[j]],
                                        ne_all.at[j], semAdj))
        adj_cps.append(pltpu.async_copy(adj_rel_hbm.at[adjidx_all.at[j]],
                                        nr_all.at[j], semAdj))

    def fire_rows(c, buf, sem):
        for j in range(JC):
            pltpu.async_copy(ent_hbm.at[uandi_all.at[c * JC + j]],
                             rows_v.at[buf, pl.ds(j * 128, 128)], sem)
        ustage[buf] = u_all[c // 8, pl.ds((c % 8) * 16, 16)]
        pltpu.async_copy(usr_hbm.at[ustage.at[buf]], user_v.at[buf], sem)

    def wait_rows(c, buf, sem):
        for j in range(JC):
            pltpu.make_async_copy(ent_hbm.at[uandi_all.at[c * JC + j]],
                                  rows_v.at[buf, pl.ds(j * 128, 128)],
                                  sem).wait()
        pltpu.make_async_copy(usr_hbm.at[ustage.at[buf]],
                              user_v.at[buf], sem).wait()

    def compute(buf):
        def belem(b, carry2):
            accs = [[jnp.zeros((16,), _f32) for _ in range(8)]
                    for _ in range(3)]
            for c in range(K):
                f0, f1, f2 = 3 * c, 3 * c + 1, 3 * c + 2
                w0 = wb_v[f0 // 8, pl.ds((f0 % 8) * 16, 16)]
                w1 = wb_v[f1 // 8, pl.ds((f1 % 8) * 16, 16)]
                w2 = wb_v[f2 // 8, pl.ds((f2 % 8) * 16, 16)]
                for dc in range(8):
                    r = rows_v[buf, b * K + c, pl.ds(dc * 16, 16)]
                    accs[0][dc] += r * w0
                    accs[1][dc] += r * w1
                    accs[2][dc] += r * w2
            for dc in range(8):
                ur = user_v[buf, b, pl.ds(dc * 16, 16)]
                for k in range(3):
                    t_v[buf, k, b, pl.ds(dc * 16, 16)] = accs[k][dc] * ur
            return carry2

        lax.fori_loop(0, C, belem, 0)

    def fire_out(c, buf, sem):
        base = pl.multiple_of(wbase + c * C, C)
        for k in range(3):
            pltpu.async_copy(t_v.at[buf, k], t_out.at[k, pl.ds(base, C)], sem)

    def wait_out(c, buf, sem):
        base = pl.multiple_of(wbase + c * C, C)
        for k in range(3):
            pltpu.make_async_copy(t_v.at[buf, k],
                                  t_out.at[k, pl.ds(base, C)], sem).wait()

    fire_rows(0, 0, semA)

    def pipe(i, carry):
        cA = i * 2
        cB = i * 2 + 1
        fire_rows(cB, 1, semB)

        @pl.when(i > 0)
        def _():
            wait_out(cA - 2, 0, semOutA)

        wait_rows(cA, 0, semA)
        compute(0)
        fire_out(cA, 0, semOutA)

        @pl.when(i < N_CHUNKS // 2 - 1)
        def _():
            fire_rows(cB + 1, 0, semA)

        @pl.when(i > 0)
        def _():
            wait_out(cB - 2, 1, semOutB)

        wait_rows(cB, 1, semB)
        compute(1)
        fire_out(cB, 1, semOutB)
        return carry

    lax.fori_loop(0, N_CHUNKS // 2, pipe, 0)
    wait_out(N_CHUNKS - 2, 0, semOutA)
    wait_out(N_CHUNKS - 1, 1, semOutB)
    for cp in adj_cps:
        cp.wait()
    for j in range(PER_W * K // 128):
        pltpu.sync_copy(ne_all.at[j],
                        ne_out.at[pl.ds(wbase * K + j * 128, 128)])
        pltpu.sync_copy(nr_all.at[j],
                        nr_out.at[pl.ds(wbase * K + j * 128, 128)])


def _sc_stage1(u, adj_idx, uandi_flat, usr_table, ent_table, adj_ent_flat,
               adj_rel_flat, wb):
    mesh = plsc.VectorSubcoreMesh(core_axis_name="c", subcore_axis_name="s")
    kfn = pl.kernel(
        _sc1_body,
        out_type=(
            jax.ShapeDtypeStruct((3, B, D), _f32),
            jax.ShapeDtypeStruct((B * K,), _i32),
            jax.ShapeDtypeStruct((B * K,), _i32),
        ),
        mesh=mesh,
        scratch_types=[
            pltpu.VMEM((PER_W // 128, 128), _i32),
            pltpu.VMEM((PER_W * K // 128, 128), _i32),
            pltpu.VMEM((PER_W * K // 128, 128), _i32),
            pltpu.VMEM((PER_W * K // 128, 128), _i32),
            pltpu.VMEM((PER_W * K // 128, 128), _i32),
            pltpu.VMEM((2, 16), _i32),
            pltpu.VMEM((2, C * K, D), _f32),
            pltpu.VMEM((2, C, D), _f32),
            pltpu.VMEM((6, 128), _f32),
            pltpu.VMEM((2, 3, C, D), _f32),
            pltpu.SemaphoreType.DMA,
            pltpu.SemaphoreType.DMA,
            pltpu.SemaphoreType.DMA,
            pltpu.SemaphoreType.DMA,
            pltpu.SemaphoreType.DMA,
        ],
    )
    return kfn(u, adj_idx, uandi_flat, usr_table, ent_table, adj_ent_flat,
               adj_rel_flat, wb)


def _sc3_body(v_hbm, ne_hbm, sc_hbm, ent_hbm, agg_out,
              v_all, ne_all, vstage, rows_v, self_v, sc_v, out_v,
              semA, semB, semOutA, semOutB):
    wid = lax.axis_index("s") * NC + lax.axis_index("c")
    wbase = pl.multiple_of(wid * PER_W, PER_W)
    for j in range(PER_W // 128):
        pltpu.sync_copy(v_hbm.at[pl.ds(wbase + j * 128, 128)], v_all.at[j])
    for j in range(PER_W * K // 128):
        pltpu.sync_copy(ne_hbm.at[pl.ds(wbase * K + j * 128, 128)],
                        ne_all.at[j])

    def fire_rows(c, buf, sem):
        for j in range(JC):
            pltpu.async_copy(ent_hbm.at[ne_all.at[c * JC + j]],
                             rows_v.at[buf, pl.ds(j * 128, 128)], sem)
        vstage[buf] = v_all[c // 8, pl.ds((c % 8) * 16, 16)]
        pltpu.async_copy(ent_hbm.at[vstage.at[buf]], self_v.at[buf], sem)
        base = pl.multiple_of(wbase + c * C, C)
        for h in range(2):
            pltpu.async_copy(sc_hbm.at[h, pl.ds(base, C)],
                             sc_v.at[buf, h], sem)

    def wait_rows(c, buf, sem):
        for j in range(JC):
            pltpu.make_async_copy(ent_hbm.at[ne_all.at[c * JC + j]],
                                  rows_v.at[buf, pl.ds(j * 128, 128)],
                                  sem).wait()
        pltpu.make_async_copy(ent_hbm.at[vstage.at[buf]],
                              self_v.at[buf], sem).wait()
        base = pl.multiple_of(wbase + c * C, C)
        for h in range(2):
            pltpu.make_async_copy(sc_hbm.at[h, pl.ds(base, C)],
                                  sc_v.at[buf, h], sem).wait()

    def compute(buf):
        def belem(b, carry2):
            accs = [self_v[buf, b, pl.ds(dc * 16, 16)] for dc in range(8)]
            for k in range(K):
                s = sc_v[buf, k // 8, b, pl.ds((k % 8) * 16, 16)]
                for dc in range(8):
                    accs[dc] += rows_v[buf, b * K + k, pl.ds(dc * 16, 16)] * s
            for dc in range(8):
                out_v[buf, b, pl.ds(dc * 16, 16)] = accs[dc]
            return carry2

        lax.fori_loop(0, C, belem, 0)

    def fire_out(c, buf, sem):
        base = pl.multiple_of(wbase + c * C, C)
        pltpu.async_copy(out_v.at[buf], agg_out.at[pl.ds(base, C)], sem)

    def wait_out(c, buf, sem):
        base = pl.multiple_of(wbase + c * C, C)
        pltpu.make_async_copy(out_v.at[buf],
                              agg_out.at[pl.ds(base, C)], sem).wait()

    fire_rows(0, 0, semA)

    def pipe(i, carry):
        cA = i * 2
        cB = i * 2 + 1
        fire_rows(cB, 1, semB)

        @pl.when(i > 0)
        def _():
            wait_out(cA - 2, 0, semOutA)

        wait_rows(cA, 0, semA)
        compute(0)
        fire_out(cA, 0, semOutA)

        @pl.when(i < N_CHUNKS // 2 - 1)
        def _():
            fire_rows(cB + 1, 0, semA)

        @pl.when(i > 0)
        def _():
            wait_out(cB - 2, 1, semOutB)

        wait_rows(cB, 1, semB)
        compute(1)
        fire_out(cB, 1, semOutB)
        return carry

    lax.fori_loop(0, N_CHUNKS // 2, pipe, 0)
    wait_out(N_CHUNKS - 2, 0, semOutA)
    wait_out(N_CHUNKS - 1, 1, semOutB)


def _sc_stage3(v, ne_flat, scores, ent_table):
    mesh = plsc.VectorSubcoreMesh(core_axis_name="c", subcore_axis_name="s")
    kfn = pl.kernel(
        _sc3_body,
        out_type=jax.ShapeDtypeStruct((B, D), _f32),
        mesh=mesh,
        scratch_types=[
            pltpu.VMEM((PER_W // 128, 128), _i32),
            pltpu.VMEM((PER_W * K // 128, 128), _i32),
            pltpu.VMEM((2, 16), _i32),
            pltpu.VMEM((2, C * K, D), _f32),
            pltpu.VMEM((2, C, D), _f32),
            pltpu.VMEM((2, 2, C, D), _f32),
            pltpu.VMEM((2, C, D), _f32),
            pltpu.SemaphoreType.DMA,
            pltpu.SemaphoreType.DMA,
            pltpu.SemaphoreType.DMA,
            pltpu.SemaphoreType.DMA,
        ],
    )
    return kfn(v, ne_flat, scores, ent_table)


R2 = 512


def _tc2_body(t_ref, nbr_ref, rel_ref, cb_ref, ue_ref, sc_ref):
    t0 = t_ref[0]
    t1 = t_ref[1]
    t2 = t_ref[2]
    z = jnp.zeros((R2, 1), _f32)
    ue = (jnp.concatenate([z, t0[:, :-1]], axis=1) + t1
          + jnp.concatenate([t2[:, 1:], z], axis=1) + cb_ref[0, 0])
    ue_ref[...] = ue
    p = lax.dot_general(ue, rel_ref[...], (((1,), (1,)), ((), ())),
                        preferred_element_type=_f32)
    nbr = nbr_ref[...]
    iota = lax.broadcasted_iota(_i32, (R2, NUM_REL), 1)
    cols = []
    for k in range(K):
        masked = jnp.where(nbr[:, k:k + 1] == iota, p, 0.0)
        cols.append(jnp.sum(masked, axis=1, keepdims=True))
    scores = jnp.concatenate(cols, axis=1)
    m = jnp.max(scores, axis=1, keepdims=True)
    e = jnp.exp(scores - m)
    sm = e / jnp.sum(e, axis=1, keepdims=True)
    lane_k = lax.broadcasted_iota(_i32, (K, D), 1) // 16
    row_k = lax.broadcasted_iota(_i32, (K, D), 0)
    e0 = (row_k == lane_k).astype(_f32)
    e1 = (row_k == lane_k + 8).astype(_f32)
    sc_ref[0] = jnp.dot(sm, e0, preferred_element_type=_f32)
    sc_ref[1] = jnp.dot(sm, e1, preferred_element_type=_f32)


def _tc_stage2(t, nbr_rel, rel_table, conv_b):
    grid = (B // R2,)
    return pl.pallas_call(
        _tc2_body,
        grid=grid,
        in_specs=[
            pl.BlockSpec((3, R2, D), lambda i: (0, i, 0)),
            pl.BlockSpec((R2, K), lambda i: (i, 0)),
            pl.BlockSpec((NUM_REL, D), lambda i: (0, 0)),
            pl.BlockSpec((1, 1), lambda i: (0, 0)),
        ],
        out_specs=[
            pl.BlockSpec((R2, D), lambda i: (i, 0)),
            pl.BlockSpec((2, R2, D), lambda i: (0, i, 0)),
        ],
        out_shape=[
            jax.ShapeDtypeStruct((B, D), _f32),
            jax.ShapeDtypeStruct((2, B, D), _f32),
        ],
    )(t, nbr_rel, rel_table, conv_b)


R4 = 512


def _tc4_body(agg_ref, ue_ref, w_ref, b_ref, out_ref):
    item = jnp.tanh(
        jnp.dot(agg_ref[...], w_ref[...], preferred_element_type=_f32)
        + b_ref[...])
    s = jnp.sum(ue_ref[...] * item, axis=1)
    out_ref[...] = 1.0 / (1.0 + jnp.exp(-s))


def _tc_stage4(agg, ue, agg_w, agg_b):
    grid = (B // R4,)
    return pl.pallas_call(
        _tc4_body,
        grid=grid,
        in_specs=[
            pl.BlockSpec((R4, D), lambda i: (i, 0)),
            pl.BlockSpec((R4, D), lambda i: (i, 0)),
            pl.BlockSpec((D, D), lambda i: (0, 0)),
            pl.BlockSpec((1, D), lambda i: (0, 0)),
        ],
        out_specs=pl.BlockSpec((R4,), lambda i: (i,)),
        out_shape=jax.ShapeDtypeStruct((B,), _f32),
    )(agg, ue, agg_w, agg_b)


@jax.jit
def kernel(u, v, uandi_adj, usr_table, ent_table, rel_table, adj_ent,
           adj_rel, conv_w, conv_b, agg_w, agg_b):
    u = u.astype(_i32)
    v = v.astype(_i32)
    uandi_flat = uandi_adj.astype(_i32).reshape(-1)
    adj_ent = adj_ent.astype(_i32)
    adj_rel = adj_rel.astype(_i32)

    w_eff = conv_w[0, :, :, 1]
    wb = jnp.broadcast_to(w_eff.reshape(6, 8, 1),
                          (6, 8, 16)).reshape(6, 128)

    adj_idx = (v[:, None] * K + jnp.arange(K, dtype=_i32)).reshape(-1)

    t, ne, nr = _sc_stage1(u, adj_idx, uandi_flat, usr_table, ent_table,
                           adj_ent.reshape(-1), adj_rel.reshape(-1), wb)
    ue, scores = _tc_stage2(t, nr.reshape(B, K), rel_table,
                            conv_b.reshape(1, 1))
    agg = _sc_stage3(v, ne, scores, ent_table)
    return _tc_stage4(agg, ue, agg_w, agg_b.reshape(1, D))


# speedup vs baseline: 1.1279x; 1.1179x over previous
import jax
import jax.numpy as jnp
from jax import lax
from jax.experimental import pallas as pl
from jax.experimental.pallas import tpu as pltpu
from jax.experimental.pallas import tpu_sc as plsc

B = 16384
D = 128
K = 16
NUM_REL = 64

NC = 2
NS = 16
NW = NC * NS
PER_W = B // NW
C = 16
N_CHUNKS = PER_W // C
JC = C * K // 128

_f32 = jnp.float32
_i32 = jnp.int32


def _sc1_body(u_hbm, adj_idx_hbm, uandi_hbm, usr_hbm, ent_hbm, adj_ent_hbm,
              adj_rel_hbm, wb_hbm, t_out, ne_out, nr_out,
              u_all, uandi_all, adjidx_all, ne_all, nr_all, ustage, rows_v,
              user_v, wb_v, t_v, semA, semB, semOutA, semOutB, semAdj):
    wid = lax.axis_index("s") * NC + lax.axis_index("c")
    wbase = pl.multiple_of(wid * PER_W, PER_W)
    pltpu.sync_copy(wb_hbm, wb_v)
    for j in range(PER_W // 128):
        pltpu.sync_copy(u_hbm.at[pl.ds(wbase + j * 128, 128)], u_all.at[j])
    for j in range(PER_W * K // 128):
        pltpu.sync_copy(uandi_hbm.at[pl.ds(wbase * K + j * 128, 128)],
                        uandi_all.at[j])
        pltpu.sync_copy(adj_idx_hbm.at[pl.ds(wbase * K + j * 128, 128)],
                        adjidx_all.at[j])
    adj_cps = []
    for j in range(PER_W * K // 128):
        adj_cps.append(pltpu.async_copy(adj_ent_hbm.at[adjidx_all.at[j]],
                                        ne_all.at[j], semAdj))
        adj_cps.append(pltpu.async_copy(adj_rel_hbm.at[adjidx_all.at[j]],
                                        nr_all.at[j], semAdj))

    def fire_rows(c, buf, sem):
        for j in range(JC):
            pltpu.async_copy(ent_hbm.at[uandi_all.at[c * JC + j]],
                             rows_v.at[buf, pl.ds(j * 128, 128)], sem)
        ustage[buf] = u_all[c // 8, pl.ds((c % 8) * 16, 16)]
        pltpu.async_copy(usr_hbm.at[ustage.at[buf]], user_v.at[buf], sem)

    def wait_rows(c, buf, sem):
        for j in range(JC):
            pltpu.make_async_copy(ent_hbm.at[uandi_all.at[c * JC + j]],
                                  rows_v.at[buf, pl.ds(j * 128, 128)],
                                  sem).wait()
        pltpu.make_async_copy(usr_hbm.at[ustage.at[buf]],
                              user_v.at[buf], sem).wait()

    def compute(buf):
        def belem(b, carry2):
            accs = [[jnp.zeros((16,), _f32) for _ in range(8)]
                    for _ in range(3)]
            for c in range(K):
                f0, f1, f2 = 3 * c, 3 * c + 1, 3 * c + 2
                w0 = wb_v[f0 // 8, pl.ds((f0 % 8) * 16, 16)]
                w1 = wb_v[f1 // 8, pl.ds((f1 % 8) * 16, 16)]
                w2 = wb_v[f2 // 8, pl.ds((f2 % 8) * 16, 16)]
                for dc in range(8):
                    r = rows_v[buf, b * K + c, pl.ds(dc * 16, 16)]
                    accs[0][dc] += r * w0
                    accs[1][dc] += r * w1
                    accs[2][dc] += r * w2
            for dc in range(8):
                ur = user_v[buf, b, pl.ds(dc * 16, 16)]
                for k in range(3):
                    t_v[buf, k, b, pl.ds(dc * 16, 16)] = accs[k][dc] * ur
            return carry2

        lax.fori_loop(0, C, belem, 0)

    def fire_out(c, buf, sem):
        base = pl.multiple_of(wbase + c * C, C)
        for k in range(3):
            pltpu.async_copy(t_v.at[buf, k], t_out.at[k, pl.ds(base, C)], sem)

    def wait_out(c, buf, sem):
        base = pl.multiple_of(wbase + c * C, C)
        for k in range(3):
            pltpu.make_async_copy(t_v.at[buf, k],
                                  t_out.at[k, pl.ds(base, C)], sem).wait()

    fire_rows(0, 0, semA)

    def pipe(i, carry):
        cA = i * 2
        cB = i * 2 + 1
        fire_rows(cB, 1, semB)

        @pl.when(i > 0)
        def _():
            wait_out(cA - 2, 0, semOutA)

        wait_rows(cA, 0, semA)
        compute(0)
        fire_out(cA, 0, semOutA)

        @pl.when(i < N_CHUNKS // 2 - 1)
        def _():
            fire_rows(cB + 1, 0, semA)

        @pl.when(i > 0)
        def _():
            wait_out(cB - 2, 1, semOutB)

        wait_rows(cB, 1, semB)
        compute(1)
        fire_out(cB, 1, semOutB)
        return carry

    lax.fori_loop(0, N_CHUNKS // 2, pipe, 0)
    wait_out(N_CHUNKS - 2, 0, semOutA)
    wait_out(N_CHUNKS - 1, 1, semOutB)
    for cp in adj_cps:
        cp.wait()
    for j in range(PER_W * K // 128):
        pltpu.sync_copy(ne_all.at[j],
                        ne_out.at[pl.ds(wbase * K + j * 128, 128)])
        pltpu.sync_copy(nr_all.at[j],
                        nr_out.at[pl.ds(wbase * K + j * 128, 128)])


def _sc_stage1(u, adj_idx, uandi_flat, usr_table, ent_table, adj_ent_flat,
               adj_rel_flat, wb):
    mesh = plsc.VectorSubcoreMesh(core_axis_name="c", subcore_axis_name="s")
    kfn = pl.kernel(
        _sc1_body,
        out_type=(
            jax.ShapeDtypeStruct((3, B, D), _f32),
            jax.ShapeDtypeStruct((B * K,), _i32),
            jax.ShapeDtypeStruct((B * K,), _i32),
        ),
        mesh=mesh,
        scratch_types=[
            pltpu.VMEM((PER_W // 128, 128), _i32),
            pltpu.VMEM((PER_W * K // 128, 128), _i32),
            pltpu.VMEM((PER_W * K // 128, 128), _i32),
            pltpu.VMEM((PER_W * K // 128, 128), _i32),
            pltpu.VMEM((PER_W * K // 128, 128), _i32),
            pltpu.VMEM((2, 16), _i32),
            pltpu.VMEM((2, C * K, D), _f32),
            pltpu.VMEM((2, C, D), _f32),
            pltpu.VMEM((6, 128), _f32),
            pltpu.VMEM((2, 3, C, D), _f32),
            pltpu.SemaphoreType.DMA,
            pltpu.SemaphoreType.DMA,
            pltpu.SemaphoreType.DMA,
            pltpu.SemaphoreType.DMA,
            pltpu.SemaphoreType.DMA,
        ],
    )
    return kfn(u, adj_idx, uandi_flat, usr_table, ent_table, adj_ent_flat,
               adj_rel_flat, wb)


def _sc3_body(v_hbm, ne_hbm, sc_hbm, ent_hbm, agg_out,
              v_all, ne_all, vstage, rows_v, self_v, sc_v, out_v,
              semA, semB, semOutA, semOutB):
    wid = lax.axis_index("s") * NC + lax.axis_index("c")
    wbase = pl.multiple_of(wid * PER_W, PER_W)
    for j in range(PER_W // 128):
        pltpu.sync_copy(v_hbm.at[pl.ds(wbase + j * 128, 128)], v_all.at[j])
    for j in range(PER_W * K // 128):
        pltpu.sync_copy(ne_hbm.at[pl.ds(wbase * K + j * 128, 128)],
                        ne_all.at[j])

    def fire_rows(c, buf, sem):
        for j in range(JC):
            pltpu.async_copy(ent_hbm.at[ne_all.at[c * JC + j]],
                             rows_v.at[buf, pl.ds(j * 128, 128)], sem)
        vstage[buf] = v_all[c // 8, pl.ds((c % 8) * 16, 16)]
        pltpu.async_copy(ent_hbm.at[vstage.at[buf]], self_v.at[buf], sem)
        base = pl.multiple_of(wbase + c * C, C)
        for h in range(2):
            pltpu.async_copy(sc_hbm.at[h, pl.ds(base, C)],
                             sc_v.at[buf, h], sem)

    def wait_rows(c, buf, sem):
        for j in range(JC):
            pltpu.make_async_copy(ent_hbm.at[ne_all.at[c * JC + j]],
                                  rows_v.at[buf, pl.ds(j * 128, 128)],
                                  sem).wait()
        pltpu.make_async_copy(ent_hbm.at[vstage.at[buf]],
                              self_v.at[buf], sem).wait()
        base = pl.multiple_of(wbase + c * C, C)
        for h in range(2):
            pltpu.make_async_copy(sc_hbm.at[h, pl.ds(base, C)],
                                  sc_v.at[buf, h], sem).wait()

    def compute(buf):
        def belem(b, carry2):
            accs = [self_v[buf, b, pl.ds(dc * 16, 16)] for dc in range(8)]
            for k in range(K):
                s = sc_v[buf, k // 8, b, pl.ds((k % 8) * 16, 16)]
                for dc in range(8):
                    accs[dc] += rows_v[buf, b * K + k, pl.ds(dc * 16, 16)] * s
            for dc in range(8):
                out_v[buf, b, pl.ds(dc * 16, 16)] = accs[dc]
            return carry2

        lax.fori_loop(0, C, belem, 0)

    def fire_out(c, buf, sem):
        base = pl.multiple_of(wbase + c * C, C)
        pltpu.async_copy(out_v.at[buf], agg_out.at[pl.ds(base, C)], sem)

    def wait_out(c, buf, sem):
        base = pl.multiple_of(wbase + c * C, C)
        pltpu.make_async_copy(out_v.at[buf],
                              agg_out.at[pl.ds(base, C)], sem).wait()

    fire_rows(0, 0, semA)

    def pipe(i, carry):
        cA = i * 2
        cB = i * 2 + 1
        fire_rows(cB, 1, semB)

        @pl.when(i > 0)
        def _():
            wait_out(cA - 2, 0, semOutA)

        wait_rows(cA, 0, semA)
        compute(0)
        fire_out(cA, 0, semOutA)

        @pl.when(i < N_CHUNKS // 2 - 1)
        def _():
            fire_rows(cB + 1, 0, semA)

        @pl.when(i > 0)
        def _():
            wait_out(cB - 2, 1, semOutB)

        wait_rows(cB, 1, semB)
        compute(1)
        fire_out(cB, 1, semOutB)
        return carry

    lax.fori_loop(0, N_CHUNKS // 2, pipe, 0)
    wait_out(N_CHUNKS - 2, 0, semOutA)
    wait_out(N_CHUNKS - 1, 1, semOutB)


def _sc_stage3(v, ne_flat, scores, ent_table):
    mesh = plsc.VectorSubcoreMesh(core_axis_name="c", subcore_axis_name="s")
    kfn = pl.kernel(
        _sc3_body,
        out_type=jax.ShapeDtypeStruct((B, D), _f32),
        mesh=mesh,
        scratch_types=[
            pltpu.VMEM((PER_W // 128, 128), _i32),
            pltpu.VMEM((PER_W * K // 128, 128), _i32),
            pltpu.VMEM((2, 16), _i32),
            pltpu.VMEM((2, C * K, D), _f32),
            pltpu.VMEM((2, C, D), _f32),
            pltpu.VMEM((2, 2, C, D), _f32),
            pltpu.VMEM((2, C, D), _f32),
            pltpu.SemaphoreType.DMA,
            pltpu.SemaphoreType.DMA,
            pltpu.SemaphoreType.DMA,
            pltpu.SemaphoreType.DMA,
        ],
    )
    return kfn(v, ne_flat, scores, ent_table)


R2 = 512


def _tc2_body(t_ref, nbr_ref, rel_ref, cb_ref, ue_ref, sc_ref):
    t0 = t_ref[0]
    t1 = t_ref[1]
    t2 = t_ref[2]
    z = jnp.zeros((R2, 1), _f32)
    ue = (jnp.concatenate([z, t0[:, :-1]], axis=1) + t1
          + jnp.concatenate([t2[:, 1:], z], axis=1) + cb_ref[0, 0])
    ue_ref[...] = ue
    p = lax.dot_general(ue, rel_ref[...], (((1,), (1,)), ((), ())),
                        preferred_element_type=_f32)
    nbr = nbr_ref[...]
    iota = lax.broadcasted_iota(_i32, (R2, NUM_REL), 1)
    col_i = lax.broadcasted_iota(_i32, (NUM_REL, K), 1)
    row_b = lax.broadcasted_iota(_i32, (K, NUM_REL), 0)
    nbr_f = nbr.astype(_f32)
    iota_f = iota.astype(_f32)
    scores = jnp.zeros((R2, K), _f32)
    for k in range(K):
        bmat = (row_b == k).astype(_f32)
        bc = jnp.dot(nbr_f, bmat, preferred_element_type=_f32)
        masked = jnp.where(bc == iota_f, p, 0.0)
        colmat = (col_i == k).astype(_f32)
        scores = scores + jnp.dot(masked, colmat,
                                  preferred_element_type=_f32)
    m = jnp.max(scores, axis=1, keepdims=True)
    e = jnp.exp(scores - m)
    sm = e / jnp.sum(e, axis=1, keepdims=True)
    lane_k = lax.broadcasted_iota(_i32, (K, D), 1) // 16
    row_k = lax.broadcasted_iota(_i32, (K, D), 0)
    e0 = (row_k == lane_k).astype(_f32)
    e1 = (row_k == lane_k + 8).astype(_f32)
    sc_ref[0] = jnp.dot(sm, e0, preferred_element_type=_f32)
    sc_ref[1] = jnp.dot(sm, e1, preferred_element_type=_f32)


def _tc_stage2(t, nbr_rel, rel_table, conv_b):
    grid = (B // R2,)
    return pl.pallas_call(
        _tc2_body,
        grid=grid,
        in_specs=[
            pl.BlockSpec((3, R2, D), lambda i: (0, i, 0)),
            pl.BlockSpec((R2, K), lambda i: (i, 0)),
            pl.BlockSpec((NUM_REL, D), lambda i: (0, 0)),
            pl.BlockSpec((1, 1), lambda i: (0, 0)),
        ],
        out_specs=[
            pl.BlockSpec((R2, D), lambda i: (i, 0)),
            pl.BlockSpec((2, R2, D), lambda i: (0, i, 0)),
        ],
        out_shape=[
            jax.ShapeDtypeStruct((B, D), _f32),
            jax.ShapeDtypeStruct((2, B, D), _f32),
        ],
    )(t, nbr_rel, rel_table, conv_b)


R4 = 512


def _tc4_body(agg_ref, ue_ref, w_ref, b_ref, out_ref):
    item = jnp.tanh(
        jnp.dot(agg_ref[...], w_ref[...], preferred_element_type=_f32)
        + b_ref[...])
    s = jnp.sum(ue_ref[...] * item, axis=1)
    out_ref[...] = 1.0 / (1.0 + jnp.exp(-s))


def _tc_stage4(agg, ue, agg_w, agg_b):
    grid = (B // R4,)
    return pl.pallas_call(
        _tc4_body,
        grid=grid,
        in_specs=[
            pl.BlockSpec((R4, D), lambda i: (i, 0)),
            pl.BlockSpec((R4, D), lambda i: (i, 0)),
            pl.BlockSpec((D, D), lambda i: (0, 0)),
            pl.BlockSpec((1, D), lambda i: (0, 0)),
        ],
        out_specs=pl.BlockSpec((R4,), lambda i: (i,)),
        out_shape=jax.ShapeDtypeStruct((B,), _f32),
    )(agg, ue, agg_w, agg_b)


@jax.jit
def kernel(u, v, uandi_adj, usr_table, ent_table, rel_table, adj_ent,
           adj_rel, conv_w, conv_b, agg_w, agg_b):
    u = u.astype(_i32)
    v = v.astype(_i32)
    uandi_flat = uandi_adj.astype(_i32).reshape(-1)
    adj_ent = adj_ent.astype(_i32)
    adj_rel = adj_rel.astype(_i32)

    w_eff = conv_w[0, :, :, 1]
    wb = jnp.broadcast_to(w_eff.reshape(6, 8, 1),
                          (6, 8, 16)).reshape(6, 128)

    adj_idx = (v[:, None] * K + jnp.arange(K, dtype=_i32)).reshape(-1)

    t, ne, nr = _sc_stage1(u, adj_idx, uandi_flat, usr_table, ent_table,
                           adj_ent.reshape(-1), adj_rel.reshape(-1), wb)
    ue, scores = _tc_stage2(t, nr.reshape(B, K), rel_table,
                            conv_b.reshape(1, 1))
    agg = _sc_stage3(v, ne, scores, ent_table)
    return _tc_stage4(agg, ue, agg_w, agg_b.reshape(1, D))


# speedup vs baseline: 1.4456x; 1.2817x over previous
import jax
import jax.numpy as jnp
from jax import lax
from jax.experimental import pallas as pl
from jax.experimental.pallas import tpu as pltpu
from jax.experimental.pallas import tpu_sc as plsc

B = 16384
D = 128
K = 16
NUM_REL = 64

NC = 2
NS = 16
NW = NC * NS
PER_W = B // NW
C = 16
N_CHUNKS = PER_W // C
JC = C * K // 128

_f32 = jnp.float32
_i32 = jnp.int32


def _sc1_body(u_hbm, adj_idx_hbm, uandi_hbm, usr_hbm, ent_hbm, adj_ent_hbm,
              adj_rel_hbm, wb_hbm, t_out, ne_out, nr_out,
              u_all, uandi_all, adjidx_all, ne_all, nr_all, ustage, rows_v,
              user_v, wb_v, t_v, semA, semB, semOutA, semOutB, semAdj):
    wid = lax.axis_index("s") * NC + lax.axis_index("c")
    wbase = pl.multiple_of(wid * PER_W, PER_W)
    wrow_u = pl.multiple_of(wid * (PER_W // 128), PER_W // 128)
    wrow_k = pl.multiple_of(wid * (PER_W * K // 128), PER_W * K // 128)
    pltpu.sync_copy(wb_hbm, wb_v)
    pltpu.sync_copy(u_hbm.at[pl.ds(wrow_u, PER_W // 128)], u_all)
    pltpu.sync_copy(uandi_hbm.at[pl.ds(wrow_k, PER_W * K // 128)], uandi_all)
    pltpu.sync_copy(adj_idx_hbm.at[pl.ds(wrow_k, PER_W * K // 128)],
                    adjidx_all)
    adj_cps = []
    for j in range(PER_W * K // 128):
        adj_cps.append(pltpu.async_copy(adj_ent_hbm.at[adjidx_all.at[j]],
                                        ne_all.at[j], semAdj))
        adj_cps.append(pltpu.async_copy(adj_rel_hbm.at[adjidx_all.at[j]],
                                        nr_all.at[j], semAdj))

    def fire_rows(c, buf, sem):
        for j in range(JC):
            pltpu.async_copy(ent_hbm.at[uandi_all.at[c * JC + j]],
                             rows_v.at[buf, pl.ds(j * 128, 128)], sem)
        ustage[buf] = u_all[c // 8, pl.ds((c % 8) * 16, 16)]
        pltpu.async_copy(usr_hbm.at[ustage.at[buf]], user_v.at[buf], sem)

    def wait_rows(c, buf, sem):
        for j in range(JC):
            pltpu.make_async_copy(ent_hbm.at[uandi_all.at[c * JC + j]],
                                  rows_v.at[buf, pl.ds(j * 128, 128)],
                                  sem).wait()
        pltpu.make_async_copy(usr_hbm.at[ustage.at[buf]],
                              user_v.at[buf], sem).wait()

    def compute(buf):
        def belem(b, carry2):
            accs = [[jnp.zeros((16,), _f32) for _ in range(8)]
                    for _ in range(3)]
            for c in range(K):
                f0, f1, f2 = 3 * c, 3 * c + 1, 3 * c + 2
                w0 = wb_v[f0 // 8, pl.ds((f0 % 8) * 16, 16)]
                w1 = wb_v[f1 // 8, pl.ds((f1 % 8) * 16, 16)]
                w2 = wb_v[f2 // 8, pl.ds((f2 % 8) * 16, 16)]
                for dc in range(8):
                    r = rows_v[buf, b * K + c, pl.ds(dc * 16, 16)]
                    accs[0][dc] += r * w0
                    accs[1][dc] += r * w1
                    accs[2][dc] += r * w2
            for dc in range(8):
                ur = user_v[buf, b, pl.ds(dc * 16, 16)]
                for k in range(3):
                    t_v[buf, k, b, pl.ds(dc * 16, 16)] = accs[k][dc] * ur
            return carry2

        lax.fori_loop(0, C, belem, 0)

    def fire_out(c, buf, sem):
        base = pl.multiple_of(wbase + c * C, C)
        for k in range(3):
            pltpu.async_copy(t_v.at[buf, k], t_out.at[k, pl.ds(base, C)], sem)

    def wait_out(c, buf, sem):
        base = pl.multiple_of(wbase + c * C, C)
        for k in range(3):
            pltpu.make_async_copy(t_v.at[buf, k],
                                  t_out.at[k, pl.ds(base, C)], sem).wait()

    fire_rows(0, 0, semA)

    def pipe(i, carry):
        cA = i * 2
        cB = i * 2 + 1
        fire_rows(cB, 1, semB)

        @pl.when(i > 0)
        def _():
            wait_out(cA - 2, 0, semOutA)

        wait_rows(cA, 0, semA)
        compute(0)
        fire_out(cA, 0, semOutA)

        @pl.when(i < N_CHUNKS // 2 - 1)
        def _():
            fire_rows(cB + 1, 0, semA)

        @pl.when(i > 0)
        def _():
            wait_out(cB - 2, 1, semOutB)

        wait_rows(cB, 1, semB)
        compute(1)
        fire_out(cB, 1, semOutB)
        return carry

    lax.fori_loop(0, N_CHUNKS // 2, pipe, 0)
    wait_out(N_CHUNKS - 2, 0, semOutA)
    wait_out(N_CHUNKS - 1, 1, semOutB)
    for cp in adj_cps:
        cp.wait()
    pltpu.sync_copy(ne_all, ne_out.at[pl.ds(wrow_k, PER_W * K // 128)])
    pltpu.sync_copy(nr_all, nr_out.at[pl.ds(wrow_k, PER_W * K // 128)])


def _sc_stage1(u, adj_idx, uandi_flat, usr_table, ent_table, adj_ent_flat,
               adj_rel_flat, wb):
    mesh = plsc.VectorSubcoreMesh(core_axis_name="c", subcore_axis_name="s")
    kfn = pl.kernel(
        _sc1_body,
        out_type=(
            jax.ShapeDtypeStruct((3, B, D), _f32),
            jax.ShapeDtypeStruct((B * K // 128, 128), _i32),
            jax.ShapeDtypeStruct((B * K // 128, 128), _i32),
        ),
        mesh=mesh,
        scratch_types=[
            pltpu.VMEM((PER_W // 128, 128), _i32),
            pltpu.VMEM((PER_W * K // 128, 128), _i32),
            pltpu.VMEM((PER_W * K // 128, 128), _i32),
            pltpu.VMEM((PER_W * K // 128, 128), _i32),
            pltpu.VMEM((PER_W * K // 128, 128), _i32),
            pltpu.VMEM((2, 16), _i32),
            pltpu.VMEM((2, C * K, D), _f32),
            pltpu.VMEM((2, C, D), _f32),
            pltpu.VMEM((6, 128), _f32),
            pltpu.VMEM((2, 3, C, D), _f32),
            pltpu.SemaphoreType.DMA,
            pltpu.SemaphoreType.DMA,
            pltpu.SemaphoreType.DMA,
            pltpu.SemaphoreType.DMA,
            pltpu.SemaphoreType.DMA,
        ],
    )
    return kfn(u, adj_idx, uandi_flat, usr_table, ent_table, adj_ent_flat,
               adj_rel_flat, wb)


def _sc3_body(v_hbm, ne_hbm, sc_hbm, ent_hbm, agg_out,
              v_all, ne_all, vstage, rows_v, self_v, sc_v, out_v,
              semA, semB, semOutA, semOutB):
    wid = lax.axis_index("s") * NC + lax.axis_index("c")
    wbase = pl.multiple_of(wid * PER_W, PER_W)
    wrow_u = pl.multiple_of(wid * (PER_W // 128), PER_W // 128)
    wrow_k = pl.multiple_of(wid * (PER_W * K // 128), PER_W * K // 128)
    pltpu.sync_copy(v_hbm.at[pl.ds(wrow_u, PER_W // 128)], v_all)
    pltpu.sync_copy(ne_hbm.at[pl.ds(wrow_k, PER_W * K // 128)], ne_all)

    def fire_rows(c, buf, sem):
        for j in range(JC):
            pltpu.async_copy(ent_hbm.at[ne_all.at[c * JC + j]],
                             rows_v.at[buf, pl.ds(j * 128, 128)], sem)
        vstage[buf] = v_all[c // 8, pl.ds((c % 8) * 16, 16)]
        pltpu.async_copy(ent_hbm.at[vstage.at[buf]], self_v.at[buf], sem)
        base = pl.multiple_of(wbase + c * C, C)
        for h in range(2):
            pltpu.async_copy(sc_hbm.at[h, pl.ds(base, C)],
                             sc_v.at[buf, h], sem)

    def wait_rows(c, buf, sem):
        for j in range(JC):
            pltpu.make_async_copy(ent_hbm.at[ne_all.at[c * JC + j]],
                                  rows_v.at[buf, pl.ds(j * 128, 128)],
                                  sem).wait()
        pltpu.make_async_copy(ent_hbm.at[vstage.at[buf]],
                              self_v.at[buf], sem).wait()
        base = pl.multiple_of(wbase + c * C, C)
        for h in range(2):
            pltpu.make_async_copy(sc_hbm.at[h, pl.ds(base, C)],
                                  sc_v.at[buf, h], sem).wait()

    def compute(buf):
        def belem(b, carry2):
            accs = [self_v[buf, b, pl.ds(dc * 16, 16)] for dc in range(8)]
            for k in range(K):
                s = sc_v[buf, k // 8, b, pl.ds((k % 8) * 16, 16)]
                for dc in range(8):
                    accs[dc] += rows_v[buf, b * K + k, pl.ds(dc * 16, 16)] * s
            for dc in range(8):
                out_v[buf, b, pl.ds(dc * 16, 16)] = accs[dc]
            return carry2

        lax.fori_loop(0, C, belem, 0)

    def fire_out(c, buf, sem):
        base = pl.multiple_of(wbase + c * C, C)
        pltpu.async_copy(out_v.at[buf], agg_out.at[pl.ds(base, C)], sem)

    def wait_out(c, buf, sem):
        base = pl.multiple_of(wbase + c * C, C)
        pltpu.make_async_copy(out_v.at[buf],
                              agg_out.at[pl.ds(base, C)], sem).wait()

    fire_rows(0, 0, semA)

    def pipe(i, carry):
        cA = i * 2
        cB = i * 2 + 1
        fire_rows(cB, 1, semB)

        @pl.when(i > 0)
        def _():
            wait_out(cA - 2, 0, semOutA)

        wait_rows(cA, 0, semA)
        compute(0)
        fire_out(cA, 0, semOutA)

        @pl.when(i < N_CHUNKS // 2 - 1)
        def _():
            fire_rows(cB + 1, 0, semA)

        @pl.when(i > 0)
        def _():
            wait_out(cB - 2, 1, semOutB)

        wait_rows(cB, 1, semB)
        compute(1)
        fire_out(cB, 1, semOutB)
        return carry

    lax.fori_loop(0, N_CHUNKS // 2, pipe, 0)
    wait_out(N_CHUNKS - 2, 0, semOutA)
    wait_out(N_CHUNKS - 1, 1, semOutB)


def _sc_stage3(v, ne_flat, scores, ent_table):
    mesh = plsc.VectorSubcoreMesh(core_axis_name="c", subcore_axis_name="s")
    kfn = pl.kernel(
        _sc3_body,
        out_type=jax.ShapeDtypeStruct((B, D), _f32),
        mesh=mesh,
        scratch_types=[
            pltpu.VMEM((PER_W // 128, 128), _i32),
            pltpu.VMEM((PER_W * K // 128, 128), _i32),
            pltpu.VMEM((2, 16), _i32),
            pltpu.VMEM((2, C * K, D), _f32),
            pltpu.VMEM((2, C, D), _f32),
            pltpu.VMEM((2, 2, C, D), _f32),
            pltpu.VMEM((2, C, D), _f32),
            pltpu.SemaphoreType.DMA,
            pltpu.SemaphoreType.DMA,
            pltpu.SemaphoreType.DMA,
            pltpu.SemaphoreType.DMA,
        ],
    )
    return kfn(v, ne_flat, scores, ent_table)


R2 = 512


def _tc2_body(t_ref, nbr_ref, rel_ref, cb_ref, ue_ref, sc_ref):
    t0 = t_ref[0]
    t1 = t_ref[1]
    t2 = t_ref[2]
    z = jnp.zeros((R2, 1), _f32)
    ue = (jnp.concatenate([z, t0[:, :-1]], axis=1) + t1
          + jnp.concatenate([t2[:, 1:], z], axis=1) + cb_ref[0, 0])
    ue_ref[...] = ue
    p = lax.dot_general(ue, rel_ref[...], (((1,), (1,)), ((), ())),
                        preferred_element_type=_f32)
    nbr = nbr_ref[...]
    iota = lax.broadcasted_iota(_i32, (R2, NUM_REL), 1)
    col_i = lax.broadcasted_iota(_i32, (NUM_REL, K), 1)
    row_b = lax.broadcasted_iota(_i32, (K, NUM_REL), 0)
    nbr_f = nbr.astype(_f32)
    iota_f = iota.astype(_f32)
    scores = jnp.zeros((R2, K), _f32)
    for k in range(K):
        bmat = (row_b == k).astype(_f32)
        bc = jnp.dot(nbr_f, bmat, preferred_element_type=_f32)
        masked = jnp.where(bc == iota_f, p, 0.0)
        colmat = (col_i == k).astype(_f32)
        scores = scores + jnp.dot(masked, colmat,
                                  preferred_element_type=_f32)
    m = jnp.max(scores, axis=1, keepdims=True)
    e = jnp.exp(scores - m)
    sm = e / jnp.sum(e, axis=1, keepdims=True)
    lane_k = lax.broadcasted_iota(_i32, (K, D), 1) // 16
    row_k = lax.broadcasted_iota(_i32, (K, D), 0)
    e0 = (row_k == lane_k).astype(_f32)
    e1 = (row_k == lane_k + 8).astype(_f32)
    sc_ref[0] = jnp.dot(sm, e0, preferred_element_type=_f32)
    sc_ref[1] = jnp.dot(sm, e1, preferred_element_type=_f32)


def _tc_stage2(t, nbr_rel, rel_table, conv_b):
    grid = (B // R2,)
    return pl.pallas_call(
        _tc2_body,
        grid=grid,
        in_specs=[
            pl.BlockSpec((3, R2, D), lambda i: (0, i, 0)),
            pl.BlockSpec((R2, K), lambda i: (i, 0)),
            pl.BlockSpec((NUM_REL, D), lambda i: (0, 0)),
            pl.BlockSpec((1, 1), lambda i: (0, 0)),
        ],
        out_specs=[
            pl.BlockSpec((R2, D), lambda i: (i, 0)),
            pl.BlockSpec((2, R2, D), lambda i: (0, i, 0)),
        ],
        out_shape=[
            jax.ShapeDtypeStruct((B, D), _f32),
            jax.ShapeDtypeStruct((2, B, D), _f32),
        ],
    )(t, nbr_rel, rel_table, conv_b)


R4 = 512


def _tc4_body(agg_ref, ue_ref, w_ref, b_ref, out_ref):
    item = jnp.tanh(
        jnp.dot(agg_ref[...], w_ref[...], preferred_element_type=_f32)
        + b_ref[...])
    s = jnp.sum(ue_ref[...] * item, axis=1)
    out_ref[...] = 1.0 / (1.0 + jnp.exp(-s))


def _tc_stage4(agg, ue, agg_w, agg_b):
    grid = (B // R4,)
    return pl.pallas_call(
        _tc4_body,
        grid=grid,
        in_specs=[
            pl.BlockSpec((R4, D), lambda i: (i, 0)),
            pl.BlockSpec((R4, D), lambda i: (i, 0)),
            pl.BlockSpec((D, D), lambda i: (0, 0)),
            pl.BlockSpec((1, D), lambda i: (0, 0)),
        ],
        out_specs=pl.BlockSpec((R4,), lambda i: (i,)),
        out_shape=jax.ShapeDtypeStruct((B,), _f32),
    )(agg, ue, agg_w, agg_b)


@jax.jit
def kernel(u, v, uandi_adj, usr_table, ent_table, rel_table, adj_ent,
           adj_rel, conv_w, conv_b, agg_w, agg_b):
    u = u.astype(_i32).reshape(B // 128, 128)
    v = v.astype(_i32)
    v2 = v.reshape(B // 128, 128)
    uandi_flat = uandi_adj.astype(_i32).reshape(B * K // 128, 128)
    adj_ent = adj_ent.astype(_i32)
    adj_rel = adj_rel.astype(_i32)

    w_eff = conv_w[0, :, :, 1]
    wb = jnp.broadcast_to(w_eff.reshape(6, 8, 1),
                          (6, 8, 16)).reshape(6, 128)

    adj_idx = (v[:, None] * K
               + jnp.arange(K, dtype=_i32)).reshape(B * K // 128, 128)

    t, ne, nr = _sc_stage1(u, adj_idx, uandi_flat, usr_table, ent_table,
                           adj_ent.reshape(-1), adj_rel.reshape(-1), wb)
    ue, scores = _tc_stage2(t, nr.reshape(B, K), rel_table,
                            conv_b.reshape(1, 1))
    agg = _sc_stage3(v2, ne, scores, ent_table)
    return _tc_stage4(agg, ue, agg_w, agg_b.reshape(1, D))
